# Initial kernel scaffold; baseline (speedup 1.0000x reference)
import jax
import jax.numpy as jnp
from jax.experimental import pallas as pl


def kernel(h, edge_index0, edge_index1, fc0, attn_l0, attn_r0, bias0, fc1, attn_l1, attn_r1, bias1, Ws1, bs1, Ws2, Wp, bp):
    raise NotImplementedError("write your pallas kernel here")


# speedup vs baseline: 54.2147x; 54.2147x over previous
import functools

import jax
import jax.numpy as jnp
from jax import lax
from jax.experimental import pallas as pl
from jax.experimental.pallas import tpu as pltpu
from jax.experimental.pallas import tpu_sc as plsc

N = 10000
F = 128
H = 8
D = 8
HD = 64
SEM = 128
OUT = 16
NP = 10240
E = 320000
EP = 327680
NTILES = 32
EPT = EP // NTILES
CH = 128
NCHUNK = EPT // CH
RW = 80
BN = 640
GRID = NP // BN


def _tc_pre(h_ref, fc0_ref, al0_ref, ar0_ref, fc1_ref, al1_ref, ar1_ref,
            f0_ref, f1_ref, er0_ref, er1_ref, elm0_ref, elm1_ref):
    blk = h_ref[...]
    first = pl.program_id(0) == 0
    for (fc, al, ar, f_out, er_out, elm) in (
            (fc0_ref, al0_ref, ar0_ref, f0_ref, er0_ref, elm0_ref),
            (fc1_ref, al1_ref, ar1_ref, f1_ref, er1_ref, elm1_ref)):
        feat = jnp.dot(blk, fc[...], preferred_element_type=jnp.float32)
        el = jnp.dot(feat, al[...], preferred_element_type=jnp.float32)
        er = jnp.dot(feat, ar[...], preferred_element_type=jnp.float32)
        zeros8 = jnp.zeros((BN, 8), jnp.float32)
        f_out[...] = jnp.concatenate([feat, el, zeros8], axis=1)
        er_out[...] = jnp.concatenate([er, zeros8], axis=1)
        m = jnp.max(el, axis=0, keepdims=True)

        @pl.when(first)
        def _():
            elm[...] = m

        @pl.when(jnp.logical_not(first))
        def _():
            elm[...] = jnp.maximum(elm[...], m)


_pre_call = pl.pallas_call(
    _tc_pre,
    grid=(GRID,),
    in_specs=[
        pl.BlockSpec((BN, F), lambda i: (i, 0)),
        pl.BlockSpec((F, HD), lambda i: (0, 0)),
        pl.BlockSpec((HD, H), lambda i: (0, 0)),
        pl.BlockSpec((HD, H), lambda i: (0, 0)),
        pl.BlockSpec((F, HD), lambda i: (0, 0)),
        pl.BlockSpec((HD, H), lambda i: (0, 0)),
        pl.BlockSpec((HD, H), lambda i: (0, 0)),
    ],
    out_specs=[
        pl.BlockSpec((BN, RW), lambda i: (i, 0)),
        pl.BlockSpec((BN, RW), lambda i: (i, 0)),
        pl.BlockSpec((BN, 16), lambda i: (i, 0)),
        pl.BlockSpec((BN, 16), lambda i: (i, 0)),
        pl.BlockSpec((1, 8), lambda i: (0, 0)),
        pl.BlockSpec((1, 8), lambda i: (0, 0)),
    ],
    out_shape=[
        jax.ShapeDtypeStruct((NP, RW), jnp.float32),
        jax.ShapeDtypeStruct((NP, RW), jnp.float32),
        jax.ShapeDtypeStruct((NP, 16), jnp.float32),
        jax.ShapeDtypeStruct((NP, 16), jnp.float32),
        jax.ShapeDtypeStruct((1, 8), jnp.float32),
        jax.ShapeDtypeStruct((1, 8), jnp.float32),
    ],
)


def _tc_erc(er0_ref, er1_ref, elm0_ref, elm1_ref, erc0_ref, erc1_ref):
    for (er_ref, elm_ref, erc_ref) in ((er0_ref, elm0_ref, erc0_ref),
                                       (er1_ref, elm1_ref, erc1_ref)):
        er = er_ref[:, 0:8]
        t = elm_ref[...] + er
        c = jnp.where(t > 0, t, 0.2 * t)
        erc_ref[...] = jnp.concatenate([er, c], axis=1)


_erc_call = pl.pallas_call(
    _tc_erc,
    grid=(GRID,),
    in_specs=[
        pl.BlockSpec((BN, 16), lambda i: (i, 0)),
        pl.BlockSpec((BN, 16), lambda i: (i, 0)),
        pl.BlockSpec((1, 8), lambda i: (0, 0)),
        pl.BlockSpec((1, 8), lambda i: (0, 0)),
    ],
    out_specs=[
        pl.BlockSpec((BN, 16), lambda i: (i, 0)),
        pl.BlockSpec((BN, 16), lambda i: (i, 0)),
    ],
    out_shape=[
        jax.ShapeDtypeStruct((NP, 16), jnp.float32),
        jax.ShapeDtypeStruct((NP, 16), jnp.float32),
    ],
)


def _dyn_gather(x, idx):
    return lax.gather(
        x, idx[:, None],
        lax.GatherDimensionNumbers(offset_dims=(), collapsed_slice_dims=(0,),
                                   start_index_map=(0,)),
        (1,), mode=lax.GatherScatterMode.PROMISE_IN_BOUNDS)


_sc_mesh = plsc.VectorSubcoreMesh(core_axis_name="c", subcore_axis_name="s")


@functools.partial(
    pl.kernel,
    mesh=_sc_mesh,
    compiler_params=pltpu.CompilerParams(use_tc_tiling_on_sc=False),
    out_type=[
        jax.ShapeDtypeStruct((2 * NP, RW), jnp.float32),
        jax.ShapeDtypeStruct((2 * NP, RW), jnp.float32),
    ],
    scratch_types=[
        pltpu.VMEM((CH,), jnp.int32),
        pltpu.VMEM((CH,), jnp.int32),
        pltpu.VMEM((CH, RW), jnp.float32),
        pltpu.VMEM((CH, 16), jnp.float32),
        pltpu.VMEM((CH, RW), jnp.float32),
        pltpu.VMEM_SHARED((NP, RW), jnp.float32),
        pltpu.SemaphoreType.DMA,
        pltpu.SemaphoreType.DMA,
    ],
)
def _sc_edge(f0, erc0, src0, dst0, f1, erc1, src1, dst1,
             out0, out1, sbuf, dbuf, gbuf, ebuf, zbuf, accum, sem_f, sem_e):
    cid = lax.axis_index("c")
    sid = lax.axis_index("s")
    iota = lax.iota(jnp.int32, 16)
    zero16 = jnp.zeros((16,), jnp.float32)

    def zrow(i, carry):
        for k in range(RW // 16):
            zbuf[i, pl.ds(16 * k, 16)] = zero16
        return carry

    lax.fori_loop(0, CH, zrow, 0)

    tile_rows = NP // 16
    ebase0 = (cid * 16 + sid) * EPT

    for (ftab, erctab, srcv, dstv, outv) in (
            (f0, erc0, src0, dst0, out0),
            (f1, erc1, src1, dst1, out1)):
        for r in range(tile_rows // CH):
            pltpu.sync_copy(zbuf, accum.at[pl.ds(sid * tile_rows + r * CH, CH)])
        plsc.subcore_barrier()

        def chunk(j, carry):
            eb = ebase0 + j * CH
            pltpu.sync_copy(srcv.at[pl.ds(eb, CH)], sbuf)
            pltpu.sync_copy(dstv.at[pl.ds(eb, CH)], dbuf)
            cp_f = pltpu.async_copy(ftab.at[sbuf], gbuf, sem_f)
            cp_e = pltpu.async_copy(erctab.at[dbuf], ebuf, sem_e)
            cp_f.wait()
            cp_e.wait()

            def edge(i, icarry):
                a = gbuf[i, pl.ds(HD, 16)]
                b = ebuf[i, :]
                t = a + b
                lr = jnp.where(t > 0, t, 0.2 * t)
                cvec = _dyn_gather(b, 8 + (iota & 7))
                ex = jnp.exp(lr - cvec)
                gbuf[i, pl.ds(HD, 16)] = jnp.where(iota < 8, ex, 0.0)
                hi = jnp.right_shift(iota, 3)
                for k in range(4):
                    pat = _dyn_gather(ex, 2 * k + hi)
                    gbuf[i, pl.ds(16 * k, 16)] = gbuf[i, pl.ds(16 * k, 16)] * pat
                return icarry

            lax.fori_loop(0, CH, edge, 0)
            pltpu.sync_copy(gbuf, accum.at[dbuf], add=True)
            return carry

        lax.fori_loop(0, NCHUNK, chunk, 0)
        plsc.subcore_barrier()
        for r in range(tile_rows // CH):
            row0 = sid * tile_rows + r * CH
            pltpu.sync_copy(accum.at[pl.ds(row0, CH)],
                            outv.at[pl.ds(cid * NP + row0, CH)])
        plsc.subcore_barrier()


def _tc_mid(o0a_ref, o0b_ref, o1a_ref, o1b_ref, b0_ref, b1_ref,
            ws1_ref, bs1_ref, ws2_ref, z0_ref, z1_ref, s0_ref, s1_ref):
    i = pl.program_id(0)
    first = i == 0
    rows = i * BN + lax.broadcasted_iota(jnp.int32, (BN, 1), 0)
    mask = (rows < N).astype(jnp.float32)

    for (oa, ob, b_ref, z_ref, s_ref) in ((o0a_ref, o0b_ref, b0_ref, z0_ref, s0_ref),
                                          (o1a_ref, o1b_ref, b1_ref, z1_ref, s1_ref)):
        num = oa[:, 0:HD] + ob[:, 0:HD]
        den = oa[:, HD:HD + 8] + ob[:, HD:HD + 8] + 1e-9
        r = num.reshape(BN, H, D) / den[:, :, None] + b_ref[...].reshape(1, H, D)
        z = jnp.where(r > 0, r, jnp.exp(jnp.minimum(r, 0.0)) - 1.0)
        z = z.reshape(BN, HD)
        z_ref[...] = z
        t = jnp.tanh(jnp.dot(z, ws1_ref[...], preferred_element_type=jnp.float32)
                     + bs1_ref[...])
        w = jnp.sum(t * ws2_ref[...], axis=1, keepdims=True)
        s = jnp.sum(w * mask)
        sv = jnp.full((1, 8), s, jnp.float32)

        @pl.when(first)
        def _():
            s_ref[...] = sv

        @pl.when(jnp.logical_not(first))
        def _():
            s_ref[...] = s_ref[...] + sv


_mid_call = pl.pallas_call(
    _tc_mid,
    grid=(GRID,),
    in_specs=[
        pl.BlockSpec((BN, RW), lambda i: (i, 0)),
        pl.BlockSpec((BN, RW), lambda i: (i, 0)),
        pl.BlockSpec((BN, RW), lambda i: (i, 0)),
        pl.BlockSpec((BN, RW), lambda i: (i, 0)),
        pl.BlockSpec((1, HD), lambda i: (0, 0)),
        pl.BlockSpec((1, HD), lambda i: (0, 0)),
        pl.BlockSpec((HD, SEM), lambda i: (0, 0)),
        pl.BlockSpec((1, SEM), lambda i: (0, 0)),
        pl.BlockSpec((1, SEM), lambda i: (0, 0)),
    ],
    out_specs=[
        pl.BlockSpec((BN, HD), lambda i: (i, 0)),
        pl.BlockSpec((BN, HD), lambda i: (i, 0)),
        pl.BlockSpec((1, 8), lambda i: (0, 0)),
        pl.BlockSpec((1, 8), lambda i: (0, 0)),
    ],
    out_shape=[
        jax.ShapeDtypeStruct((NP, HD), jnp.float32),
        jax.ShapeDtypeStruct((NP, HD), jnp.float32),
        jax.ShapeDtypeStruct((1, 8), jnp.float32),
        jax.ShapeDtypeStruct((1, 8), jnp.float32),
    ],
)


def _tc_fin(ws_ref, z0_ref, z1_ref, wp_ref, bp_ref, out_ref):
    wn = ws_ref[...] * (1.0 / N)
    m = jnp.max(wn)
    e = jnp.exp(wn - m)
    r0 = e[0:1, 0:1]
    r1 = e[1:2, 0:1]
    tot = r0 + r1
    b0 = r0 / tot
    b1 = r1 / tot
    hout = b0 * z0_ref[...] + b1 * z1_ref[...]
    out_ref[...] = (jnp.dot(hout, wp_ref[...], preferred_element_type=jnp.float32)
                    + bp_ref[...])


_fin_call = pl.pallas_call(
    _tc_fin,
    grid=(GRID,),
    in_specs=[
        pl.BlockSpec((2, 8), lambda i: (0, 0)),
        pl.BlockSpec((BN, HD), lambda i: (i, 0)),
        pl.BlockSpec((BN, HD), lambda i: (i, 0)),
        pl.BlockSpec((HD, OUT), lambda i: (0, 0)),
        pl.BlockSpec((1, OUT), lambda i: (0, 0)),
    ],
    out_specs=pl.BlockSpec((BN, OUT), lambda i: (i, 0)),
    out_shape=jax.ShapeDtypeStruct((NP, OUT), jnp.float32),
)


def _blockdiag(a):
    eye = jnp.eye(H, dtype=a.dtype)
    return (a[:, :, None] * eye[:, None, :]).reshape(HD, H)


def _pad_idx(v):
    return jnp.concatenate([v, jnp.full((EP - E,), NP - 1, jnp.int32)])


def kernel(h, edge_index0, edge_index1, fc0, attn_l0, attn_r0, bias0,
           fc1, attn_l1, attn_r1, bias1, Ws1, bs1, Ws2, Wp, bp):
    h_pad = jnp.pad(h, ((0, NP - N), (0, 0)))
    f0, f1, er0, er1, elm0, elm1 = _pre_call(
        h_pad, fc0, _blockdiag(attn_l0), _blockdiag(attn_r0),
        fc1, _blockdiag(attn_l1), _blockdiag(attn_r1))
    erc0, erc1 = _erc_call(er0, er1, elm0, elm1)

    o0, o1 = _sc_edge(
        f0, erc0, _pad_idx(edge_index0[0]), _pad_idx(edge_index0[1]),
        f1, erc1, _pad_idx(edge_index1[0]), _pad_idx(edge_index1[1]))

    z0, z1, s0, s1 = _mid_call(
        o0[:NP], o0[NP:], o1[:NP], o1[NP:],
        bias0.reshape(1, HD), bias1.reshape(1, HD),
        Ws1, bs1.reshape(1, SEM), Ws2.reshape(1, SEM))
    ws = jnp.concatenate([s0, s1], axis=0)
    out_full = _fin_call(ws, z0, z1, Wp, bp.reshape(1, OUT))
    return out_full[:N]


# speedup vs baseline: 91.4087x; 1.6861x over previous
import functools

import jax
import jax.numpy as jnp
from jax import lax
from jax.experimental import pallas as pl
from jax.experimental.pallas import tpu as pltpu
from jax.experimental.pallas import tpu_sc as plsc

N = 10000
F = 128
H = 8
D = 8
HD = 64
SEM = 128
OUT = 16
NP = 10240
E = 320000
EP = 327680
NTILES = 32
EPT = EP // NTILES
CH = 128
NCHUNK = EPT // CH
RW = 80
BN = 640
GRID = NP // BN


def _tc_pre(h_ref, fc0_ref, al0_ref, ar0_ref, fc1_ref, al1_ref, ar1_ref,
            f0_ref, f1_ref, er0_ref, er1_ref, elm0_ref, elm1_ref):
    blk = h_ref[...]
    first = pl.program_id(0) == 0
    for (fc, al, ar, f_out, er_out, elm) in (
            (fc0_ref, al0_ref, ar0_ref, f0_ref, er0_ref, elm0_ref),
            (fc1_ref, al1_ref, ar1_ref, f1_ref, er1_ref, elm1_ref)):
        feat = jnp.dot(blk, fc[...], preferred_element_type=jnp.float32)
        el = jnp.dot(feat, al[...], preferred_element_type=jnp.float32)
        er = jnp.dot(feat, ar[...], preferred_element_type=jnp.float32)
        zeros8 = jnp.zeros((BN, 8), jnp.float32)
        f_out[...] = jnp.concatenate([feat, el, jnp.full((BN, 8), -1e30, jnp.float32)], axis=1)
        er_out[...] = jnp.concatenate([er, zeros8], axis=1)
        m = jnp.max(el, axis=0, keepdims=True)

        @pl.when(first)
        def _():
            elm[...] = m

        @pl.when(jnp.logical_not(first))
        def _():
            elm[...] = jnp.maximum(elm[...], m)


_pre_call = pl.pallas_call(
    _tc_pre,
    grid=(GRID,),
    in_specs=[
        pl.BlockSpec((BN, F), lambda i: (i, 0)),
        pl.BlockSpec((F, HD), lambda i: (0, 0)),
        pl.BlockSpec((HD, H), lambda i: (0, 0)),
        pl.BlockSpec((HD, H), lambda i: (0, 0)),
        pl.BlockSpec((F, HD), lambda i: (0, 0)),
        pl.BlockSpec((HD, H), lambda i: (0, 0)),
        pl.BlockSpec((HD, H), lambda i: (0, 0)),
    ],
    out_specs=[
        pl.BlockSpec((BN, RW), lambda i: (i, 0)),
        pl.BlockSpec((BN, RW), lambda i: (i, 0)),
        pl.BlockSpec((BN, 16), lambda i: (i, 0)),
        pl.BlockSpec((BN, 16), lambda i: (i, 0)),
        pl.BlockSpec((1, 8), lambda i: (0, 0)),
        pl.BlockSpec((1, 8), lambda i: (0, 0)),
    ],
    out_shape=[
        jax.ShapeDtypeStruct((NP, RW), jnp.float32),
        jax.ShapeDtypeStruct((NP, RW), jnp.float32),
        jax.ShapeDtypeStruct((NP, 16), jnp.float32),
        jax.ShapeDtypeStruct((NP, 16), jnp.float32),
        jax.ShapeDtypeStruct((1, 8), jnp.float32),
        jax.ShapeDtypeStruct((1, 8), jnp.float32),
    ],
)


def _tc_erc(er0_ref, er1_ref, elm0_ref, elm1_ref, erc0_ref, erc1_ref):
    for (er_ref, elm_ref, erc_ref) in ((er0_ref, elm0_ref, erc0_ref),
                                       (er1_ref, elm1_ref, erc1_ref)):
        er = er_ref[:, 0:8]
        t = elm_ref[...] + er
        c = jnp.where(t > 0, t, 0.2 * t)
        erc_ref[...] = jnp.concatenate([er, c], axis=1)


_erc_call = pl.pallas_call(
    _tc_erc,
    grid=(GRID,),
    in_specs=[
        pl.BlockSpec((BN, 16), lambda i: (i, 0)),
        pl.BlockSpec((BN, 16), lambda i: (i, 0)),
        pl.BlockSpec((1, 8), lambda i: (0, 0)),
        pl.BlockSpec((1, 8), lambda i: (0, 0)),
    ],
    out_specs=[
        pl.BlockSpec((BN, 16), lambda i: (i, 0)),
        pl.BlockSpec((BN, 16), lambda i: (i, 0)),
    ],
    out_shape=[
        jax.ShapeDtypeStruct((NP, 16), jnp.float32),
        jax.ShapeDtypeStruct((NP, 16), jnp.float32),
    ],
)


def _dyn_gather(x, idx):
    return lax.gather(
        x, idx[:, None],
        lax.GatherDimensionNumbers(offset_dims=(), collapsed_slice_dims=(0,),
                                   start_index_map=(0,)),
        (1,), mode=lax.GatherScatterMode.PROMISE_IN_BOUNDS)


_sc_mesh = plsc.VectorSubcoreMesh(core_axis_name="c", subcore_axis_name="s")


NB = 4


@functools.partial(
    pl.kernel,
    mesh=_sc_mesh,
    compiler_params=pltpu.CompilerParams(use_tc_tiling_on_sc=False),
    out_type=[
        jax.ShapeDtypeStruct((2 * NP, RW), jnp.float32),
        jax.ShapeDtypeStruct((2 * NP, RW), jnp.float32),
    ],
    scratch_types=(
        [pltpu.VMEM((NCHUNK, CH), jnp.int32)] * 2
        + [pltpu.VMEM((CH, RW), jnp.float32)] * NB
        + [pltpu.VMEM((CH, 16), jnp.float32)] * NB
        + [pltpu.VMEM((CH, RW), jnp.float32)]
        + [pltpu.VMEM_SHARED((NP, RW), jnp.float32)]
        + [pltpu.SemaphoreType.DMA] * (2 * NB)
    ),
)
def _sc_edge(f0, erc0, src0, dst0, f1, erc1, src1, dst1,
             out0, out1, *sc):
    sidx, didx = sc[0], sc[1]
    gb = sc[2:2 + NB]
    eb = sc[2 + NB:2 + 2 * NB]
    zbuf = sc[2 + 2 * NB]
    accum = sc[3 + 2 * NB]
    semf = sc[4 + 2 * NB:4 + 3 * NB]
    seme = sc[4 + 3 * NB:4 + 4 * NB]

    cid = lax.axis_index("c")
    sid = lax.axis_index("s")
    iota = lax.iota(jnp.int32, 16)
    csel = 8 + (iota & 7)
    hi = jnp.right_shift(iota, 3)
    zero16 = jnp.zeros((16,), jnp.float32)

    def zrow(i, carry):
        for k in range(RW // 16):
            zbuf[i, pl.ds(16 * k, 16)] = zero16
        return carry

    lax.fori_loop(0, CH, zrow, 0)

    tile_rows = NP // 16
    tchunk0 = (cid * 16 + sid) * NCHUNK

    for (ftab, erctab, srcv, dstv, outv) in (
            (f0, erc0, src0, dst0, out0),
            (f1, erc1, src1, dst1, out1)):
        pltpu.sync_copy(srcv.at[pl.ds(tchunk0, NCHUNK)], sidx)
        pltpu.sync_copy(dstv.at[pl.ds(tchunk0, NCHUNK)], didx)
        for r in range(tile_rows // CH):
            pltpu.sync_copy(zbuf, accum.at[pl.ds(sid * tile_rows + r * CH, CH)])
        plsc.subcore_barrier()

        def fire(slot, j):
            pltpu.async_copy(ftab.at[sidx.at[j]], gb[slot], semf[slot])
            pltpu.async_copy(erctab.at[didx.at[j]], eb[slot], seme[slot])

        for b in range(NB):
            fire(b, b)

        def edge_block(g, e):
            def edge(i, icarry):
                a = g[i, pl.ds(HD, 16)]
                bv = e[i, :]
                t = a + bv
                lr = jnp.maximum(t, 0.2 * t)
                cvec = _dyn_gather(bv, csel)
                ex = jnp.exp(lr - cvec)
                g[i, pl.ds(HD, 16)] = ex
                for k in range(4):
                    pat = _dyn_gather(ex, 2 * k + hi)
                    g[i, pl.ds(16 * k, 16)] = g[i, pl.ds(16 * k, 16)] * pat
                return icarry

            lax.fori_loop(0, CH, edge, 0, unroll=4)

        def macro(jj, carry):
            for b in range(NB):
                j = jj * NB + b
                pltpu.make_async_copy(ftab.at[sidx.at[j]], gb[b], semf[b]).wait()
                pltpu.make_async_copy(erctab.at[didx.at[j]], eb[b], seme[b]).wait()
                edge_block(gb[b], eb[b])
                pltpu.sync_copy(gb[b], accum.at[didx.at[j]], add=True)

                @pl.when(j + NB < NCHUNK)
                def _():
                    fire(b, j + NB)
            return carry

        lax.fori_loop(0, NCHUNK // NB, macro, 0)
        plsc.subcore_barrier()
        for r in range(tile_rows // CH):
            row0 = sid * tile_rows + r * CH
            pltpu.sync_copy(accum.at[pl.ds(row0, CH)],
                            outv.at[pl.ds(cid * NP + row0, CH)])
        plsc.subcore_barrier()


def _tc_mid(o0a_ref, o0b_ref, o1a_ref, o1b_ref, b0_ref, b1_ref,
            ws1_ref, bs1_ref, ws2_ref, z0_ref, z1_ref, s0_ref, s1_ref):
    i = pl.program_id(0)
    first = i == 0
    rows = i * BN + lax.broadcasted_iota(jnp.int32, (BN, 1), 0)
    mask = (rows < N).astype(jnp.float32)

    for (oa, ob, b_ref, z_ref, s_ref) in ((o0a_ref, o0b_ref, b0_ref, z0_ref, s0_ref),
                                          (o1a_ref, o1b_ref, b1_ref, z1_ref, s1_ref)):
        num = oa[:, 0:HD] + ob[:, 0:HD]
        den = oa[:, HD:HD + 8] + ob[:, HD:HD + 8] + 1e-9
        r = num.reshape(BN, H, D) / den[:, :, None] + b_ref[...].reshape(1, H, D)
        z = jnp.where(r > 0, r, jnp.exp(jnp.minimum(r, 0.0)) - 1.0)
        z = z.reshape(BN, HD)
        z_ref[...] = z
        t = jnp.tanh(jnp.dot(z, ws1_ref[...], preferred_element_type=jnp.float32)
                     + bs1_ref[...])
        w = jnp.sum(t * ws2_ref[...], axis=1, keepdims=True)
        s = jnp.sum(w * mask)
        sv = jnp.full((1, 8), s, jnp.float32)

        @pl.when(first)
        def _():
            s_ref[...] = sv

        @pl.when(jnp.logical_not(first))
        def _():
            s_ref[...] = s_ref[...] + sv


_mid_call = pl.pallas_call(
    _tc_mid,
    grid=(GRID,),
    in_specs=[
        pl.BlockSpec((BN, RW), lambda i: (i, 0)),
        pl.BlockSpec((BN, RW), lambda i: (i, 0)),
        pl.BlockSpec((BN, RW), lambda i: (i, 0)),
        pl.BlockSpec((BN, RW), lambda i: (i, 0)),
        pl.BlockSpec((1, HD), lambda i: (0, 0)),
        pl.BlockSpec((1, HD), lambda i: (0, 0)),
        pl.BlockSpec((HD, SEM), lambda i: (0, 0)),
        pl.BlockSpec((1, SEM), lambda i: (0, 0)),
        pl.BlockSpec((1, SEM), lambda i: (0, 0)),
    ],
    out_specs=[
        pl.BlockSpec((BN, HD), lambda i: (i, 0)),
        pl.BlockSpec((BN, HD), lambda i: (i, 0)),
        pl.BlockSpec((1, 8), lambda i: (0, 0)),
        pl.BlockSpec((1, 8), lambda i: (0, 0)),
    ],
    out_shape=[
        jax.ShapeDtypeStruct((NP, HD), jnp.float32),
        jax.ShapeDtypeStruct((NP, HD), jnp.float32),
        jax.ShapeDtypeStruct((1, 8), jnp.float32),
        jax.ShapeDtypeStruct((1, 8), jnp.float32),
    ],
)


def _tc_fin(ws_ref, z0_ref, z1_ref, wp_ref, bp_ref, out_ref):
    wn = ws_ref[...] * (1.0 / N)
    m = jnp.max(wn)
    e = jnp.exp(wn - m)
    r0 = e[0:1, 0:1]
    r1 = e[1:2, 0:1]
    tot = r0 + r1
    b0 = r0 / tot
    b1 = r1 / tot
    hout = b0 * z0_ref[...] + b1 * z1_ref[...]
    out_ref[...] = (jnp.dot(hout, wp_ref[...], preferred_element_type=jnp.float32)
                    + bp_ref[...])


_fin_call = pl.pallas_call(
    _tc_fin,
    grid=(GRID,),
    in_specs=[
        pl.BlockSpec((2, 8), lambda i: (0, 0)),
        pl.BlockSpec((BN, HD), lambda i: (i, 0)),
        pl.BlockSpec((BN, HD), lambda i: (i, 0)),
        pl.BlockSpec((HD, OUT), lambda i: (0, 0)),
        pl.BlockSpec((1, OUT), lambda i: (0, 0)),
    ],
    out_specs=pl.BlockSpec((BN, OUT), lambda i: (i, 0)),
    out_shape=jax.ShapeDtypeStruct((NP, OUT), jnp.float32),
)


def _blockdiag(a):
    eye = jnp.eye(H, dtype=a.dtype)
    return (a[:, :, None] * eye[:, None, :]).reshape(HD, H)


def _pad_idx(v):
    p = jnp.concatenate([v, jnp.full((EP - E,), NP - 1, jnp.int32)])
    return p.reshape(EP // CH, CH)


def kernel(h, edge_index0, edge_index1, fc0, attn_l0, attn_r0, bias0,
           fc1, attn_l1, attn_r1, bias1, Ws1, bs1, Ws2, Wp, bp):
    h_pad = jnp.pad(h, ((0, NP - N), (0, 0)))
    f0, f1, er0, er1, elm0, elm1 = _pre_call(
        h_pad, fc0, _blockdiag(attn_l0), _blockdiag(attn_r0),
        fc1, _blockdiag(attn_l1), _blockdiag(attn_r1))
    erc0, erc1 = _erc_call(er0, er1, elm0, elm1)

    o0, o1 = _sc_edge(
        f0, erc0, _pad_idx(edge_index0[0]), _pad_idx(edge_index0[1]),
        f1, erc1, _pad_idx(edge_index1[0]), _pad_idx(edge_index1[1]))

    z0, z1, s0, s1 = _mid_call(
        o0[:NP], o0[NP:], o1[:NP], o1[NP:],
        bias0.reshape(1, HD), bias1.reshape(1, HD),
        Ws1, bs1.reshape(1, SEM), Ws2.reshape(1, SEM))
    ws = jnp.concatenate([s0, s1], axis=0)
    out_full = _fin_call(ws, z0, z1, Wp, bp.reshape(1, OUT))
    return out_full[:N]


# speedup vs baseline: 104.4231x; 1.1424x over previous
import functools

import jax
import jax.numpy as jnp
from jax import lax
from jax.experimental import pallas as pl
from jax.experimental.pallas import tpu as pltpu
from jax.experimental.pallas import tpu_sc as plsc

N = 10000
F = 128
H = 8
D = 8
HD = 64
SEM = 128
OUT = 16
NP = 10240
E = 320000
EP = 327680
NTILES = 32
EPT = EP // NTILES
CH = 128
NCHUNK = EPT // CH
FBW = 96
RW = 80
BN = 640
GRID = NP // BN
NB = 4


def _tc_pre(h_ref, wb0_ref, fcl0_ref, fcr0_ref, wb1_ref, fcl1_ref, fcr1_ref,
            cb_ref, fb0_ref, fb1_ref, er0_ref, er1_ref, elm0_ref, elm1_ref):
    blk = h_ref[...]
    first = pl.program_id(0) == 0
    for (wb, fcl, fcr, fb_out, er_out, elm) in (
            (wb0_ref, fcl0_ref, fcr0_ref, fb0_ref, er0_ref, elm0_ref),
            (wb1_ref, fcl1_ref, fcr1_ref, fb1_ref, er1_ref, elm1_ref)):
        fb = jnp.dot(blk, wb[...], preferred_element_type=jnp.float32) + cb_ref[...]
        fb_out[...] = fb.astype(jnp.bfloat16)
        el = jnp.dot(blk, fcl[...], preferred_element_type=jnp.float32)
        er = jnp.dot(blk, fcr[...], preferred_element_type=jnp.float32)
        er_out[...] = jnp.concatenate([er, jnp.zeros((BN, 8), jnp.float32)], axis=1)
        m = jnp.max(el, axis=0, keepdims=True)

        @pl.when(first)
        def _():
            elm[...] = m

        @pl.when(jnp.logical_not(first))
        def _():
            elm[...] = jnp.maximum(elm[...], m)


_pre_call = pl.pallas_call(
    _tc_pre,
    grid=(GRID,),
    in_specs=[
        pl.BlockSpec((BN, F), lambda i: (i, 0)),
        pl.BlockSpec((F, FBW), lambda i: (0, 0)),
        pl.BlockSpec((F, H), lambda i: (0, 0)),
        pl.BlockSpec((F, H), lambda i: (0, 0)),
        pl.BlockSpec((F, FBW), lambda i: (0, 0)),
        pl.BlockSpec((F, H), lambda i: (0, 0)),
        pl.BlockSpec((F, H), lambda i: (0, 0)),
        pl.BlockSpec((1, FBW), lambda i: (0, 0)),
    ],
    out_specs=[
        pl.BlockSpec((BN, FBW), lambda i: (i, 0)),
        pl.BlockSpec((BN, FBW), lambda i: (i, 0)),
        pl.BlockSpec((BN, 16), lambda i: (i, 0)),
        pl.BlockSpec((BN, 16), lambda i: (i, 0)),
        pl.BlockSpec((1, 8), lambda i: (0, 0)),
        pl.BlockSpec((1, 8), lambda i: (0, 0)),
    ],
    out_shape=[
        jax.ShapeDtypeStruct((NP, FBW), jnp.bfloat16),
        jax.ShapeDtypeStruct((NP, FBW), jnp.bfloat16),
        jax.ShapeDtypeStruct((NP, 16), jnp.float32),
        jax.ShapeDtypeStruct((NP, 16), jnp.float32),
        jax.ShapeDtypeStruct((1, 8), jnp.float32),
        jax.ShapeDtypeStruct((1, 8), jnp.float32),
    ],
)


def _dyn_gather(x, idx):
    return lax.gather(
        x, idx[:, None],
        lax.GatherDimensionNumbers(offset_dims=(), collapsed_slice_dims=(0,),
                                   start_index_map=(0,)),
        (1,), mode=lax.GatherScatterMode.PROMISE_IN_BOUNDS)


_sc_mesh = plsc.VectorSubcoreMesh(core_axis_name="c", subcore_axis_name="s")


@functools.partial(
    pl.kernel,
    mesh=_sc_mesh,
    compiler_params=pltpu.CompilerParams(use_tc_tiling_on_sc=False),
    out_type=[
        jax.ShapeDtypeStruct((2 * NP, RW), jnp.float32),
        jax.ShapeDtypeStruct((2 * NP, RW), jnp.float32),
    ],
    scratch_types=(
        [pltpu.VMEM((NCHUNK, CH), jnp.int32)] * 2
        + [pltpu.VMEM((CH, FBW // 2), jnp.int32)] * NB
        + [pltpu.VMEM((CH, 16), jnp.float32)] * NB
        + [pltpu.VMEM((CH, RW), jnp.float32)]
        + [pltpu.VMEM((CH, RW), jnp.float32)]
        + [pltpu.VMEM((16,), jnp.float32)]
        + [pltpu.VMEM_SHARED((NP, RW), jnp.float32)]
        + [pltpu.SemaphoreType.DMA] * (2 * NB)
    ),
)
def _sc_edge(fb0, er0, elm0, src0, dst0, fb1, er1, elm1, src1, dst1,
             out0, out1, *sc):
    sidx, didx = sc[0], sc[1]
    gb = sc[2:2 + NB]
    eb = sc[2 + NB:2 + 2 * NB]
    mbuf = sc[2 + 2 * NB]
    zbuf = sc[3 + 2 * NB]
    elmb = sc[4 + 2 * NB]
    accum = sc[5 + 2 * NB]
    semf = sc[6 + 2 * NB:6 + 3 * NB]
    seme = sc[6 + 3 * NB:6 + 4 * NB]

    cid = lax.axis_index("c")
    sid = lax.axis_index("s")
    iota = lax.iota(jnp.int32, 16)
    lo8 = iota & 7
    hi = jnp.right_shift(iota, 3)
    zero16 = jnp.zeros((16,), jnp.float32)

    def zrow(i, carry):
        for k in range(RW // 16):
            zbuf[i, pl.ds(16 * k, 16)] = zero16
        return carry

    lax.fori_loop(0, CH, zrow, 0)

    tile_rows = NP // 16
    tchunk0 = (cid * 16 + sid) * NCHUNK

    for (ftab, ertab, elmv, srcv, dstv, outv) in (
            (fb0, er0, elm0, src0, dst0, out0),
            (fb1, er1, elm1, src1, dst1, out1)):
        pltpu.sync_copy(srcv.at[pl.ds(tchunk0, NCHUNK)], sidx)
        pltpu.sync_copy(dstv.at[pl.ds(tchunk0, NCHUNK)], didx)
        pltpu.sync_copy(elmv, elmb)
        emv = _dyn_gather(elmb[...], lo8)
        for r in range(tile_rows // CH):
            pltpu.sync_copy(zbuf, accum.at[pl.ds(sid * tile_rows + r * CH, CH)])
        plsc.subcore_barrier()

        def fire(slot, j):
            pltpu.async_copy(ftab.at[sidx.at[j]], gb[slot], semf[slot])
            pltpu.async_copy(ertab.at[didx.at[j]], eb[slot], seme[slot])

        for b in range(NB):
            fire(b, b)

        def edge_block(g, e):
            def edge(i, icarry):
                w0 = g[i, pl.ds(0, 16)]
                w1 = g[i, pl.ds(16, 16)]
                w2 = g[i, pl.ds(32, 16)]
                msk = jnp.int32(-65536)
                f0l = lax.bitcast_convert_type(lax.shift_left(w0, 16), jnp.float32)
                f0h = lax.bitcast_convert_type(w0 & msk, jnp.float32)
                f1l = lax.bitcast_convert_type(lax.shift_left(w1, 16), jnp.float32)
                f1h = lax.bitcast_convert_type(w1 & msk, jnp.float32)
                a = lax.bitcast_convert_type(lax.shift_left(w2, 16), jnp.float32)
                bv = e[i, :]
                t = a + bv
                lr = jnp.maximum(t, 0.2 * t)
                u = bv + emv
                cv = jnp.maximum(u, 0.2 * u)
                ex = jnp.exp(lr - cv)
                mbuf[i, pl.ds(HD, 16)] = ex
                mbuf[i, pl.ds(0, 16)] = f0l * _dyn_gather(ex, hi)
                mbuf[i, pl.ds(16, 16)] = f0h * _dyn_gather(ex, 2 + hi)
                mbuf[i, pl.ds(32, 16)] = f1l * _dyn_gather(ex, 4 + hi)
                mbuf[i, pl.ds(48, 16)] = f1h * _dyn_gather(ex, 6 + hi)
                return icarry

            lax.fori_loop(0, CH, edge, 0, unroll=4)

        def macro(jj, carry):
            for b in range(NB):
                j = jj * NB + b
                pltpu.make_async_copy(ftab.at[sidx.at[j]], gb[b], semf[b]).wait()
                pltpu.make_async_copy(ertab.at[didx.at[j]], eb[b], seme[b]).wait()
                edge_block(gb[b], eb[b])
                pltpu.sync_copy(mbuf, accum.at[didx.at[j]], add=True)

                @pl.when(j + NB < NCHUNK)
                def _():
                    fire(b, j + NB)
            return carry

        lax.fori_loop(0, NCHUNK // NB, macro, 0)
        plsc.subcore_barrier()
        for r in range(tile_rows // CH):
            row0 = sid * tile_rows + r * CH
            pltpu.sync_copy(accum.at[pl.ds(row0, CH)],
                            outv.at[pl.ds(cid * NP + row0, CH)])
        plsc.subcore_barrier()


def _tc_mid(o0a_ref, o0b_ref, o1a_ref, o1b_ref, b0_ref, b1_ref,
            ws1_ref, bs1_ref, ws2_ref, z0_ref, z1_ref, s0_ref, s1_ref):
    i = pl.program_id(0)
    first = i == 0
    rows = i * BN + lax.broadcasted_iota(jnp.int32, (BN, 1), 0)
    mask = (rows < N).astype(jnp.float32)

    for (oa, ob, b_ref, z_ref, s_ref) in ((o0a_ref, o0b_ref, b0_ref, z0_ref, s0_ref),
                                          (o1a_ref, o1b_ref, b1_ref, z1_ref, s1_ref)):
        num = oa[:, 0:HD] + ob[:, 0:HD]
        den = oa[:, HD:HD + 8] + ob[:, HD:HD + 8] + 1e-9
        r = num.reshape(BN, H, D) / den[:, :, None] + b_ref[...].reshape(1, H, D)
        z = jnp.where(r > 0, r, jnp.exp(jnp.minimum(r, 0.0)) - 1.0)
        z = z.reshape(BN, HD)
        z_ref[...] = z
        t = jnp.tanh(jnp.dot(z, ws1_ref[...], preferred_element_type=jnp.float32)
                     + bs1_ref[...])
        w = jnp.sum(t * ws2_ref[...], axis=1, keepdims=True)
        s = jnp.sum(w * mask)
        sv = jnp.full((1, 8), s, jnp.float32)

        @pl.when(first)
        def _():
            s_ref[...] = sv

        @pl.when(jnp.logical_not(first))
        def _():
            s_ref[...] = s_ref[...] + sv


_mid_call = pl.pallas_call(
    _tc_mid,
    grid=(GRID,),
    in_specs=[
        pl.BlockSpec((BN, RW), lambda i: (i, 0)),
        pl.BlockSpec((BN, RW), lambda i: (GRID + i, 0)),
        pl.BlockSpec((BN, RW), lambda i: (i, 0)),
        pl.BlockSpec((BN, RW), lambda i: (GRID + i, 0)),
        pl.BlockSpec((1, HD), lambda i: (0, 0)),
        pl.BlockSpec((1, HD), lambda i: (0, 0)),
        pl.BlockSpec((HD, SEM), lambda i: (0, 0)),
        pl.BlockSpec((1, SEM), lambda i: (0, 0)),
        pl.BlockSpec((1, SEM), lambda i: (0, 0)),
    ],
    out_specs=[
        pl.BlockSpec((BN, HD), lambda i: (i, 0)),
        pl.BlockSpec((BN, HD), lambda i: (i, 0)),
        pl.BlockSpec((1, 8), lambda i: (0, 0)),
        pl.BlockSpec((1, 8), lambda i: (0, 0)),
    ],
    out_shape=[
        jax.ShapeDtypeStruct((NP, HD), jnp.float32),
        jax.ShapeDtypeStruct((NP, HD), jnp.float32),
        jax.ShapeDtypeStruct((1, 8), jnp.float32),
        jax.ShapeDtypeStruct((1, 8), jnp.float32),
    ],
)


def _tc_fin(ws_ref, z0_ref, z1_ref, wp_ref, bp_ref, out_ref):
    wn = ws_ref[...] * (1.0 / N)
    m = jnp.max(wn)
    e = jnp.exp(wn - m)
    r0 = e[0:1, 0:1]
    r1 = e[1:2, 0:1]
    tot = r0 + r1
    b0 = r0 / tot
    b1 = r1 / tot
    hout = b0 * z0_ref[...] + b1 * z1_ref[...]
    out_ref[...] = (jnp.dot(hout, wp_ref[...], preferred_element_type=jnp.float32)
                    + bp_ref[...])


_fin_call = pl.pallas_call(
    _tc_fin,
    grid=(GRID,),
    in_specs=[
        pl.BlockSpec((2, 8), lambda i: (0, 0)),
        pl.BlockSpec((BN, HD), lambda i: (i, 0)),
        pl.BlockSpec((BN, HD), lambda i: (i, 0)),
        pl.BlockSpec((HD, OUT), lambda i: (0, 0)),
        pl.BlockSpec((1, OUT), lambda i: (0, 0)),
    ],
    out_specs=pl.BlockSpec((BN, OUT), lambda i: (i, 0)),
    out_shape=jax.ShapeDtypeStruct((NP, OUT), jnp.float32),
)


def _blockdiag(a):
    eye = jnp.eye(H, dtype=a.dtype)
    return (a[:, :, None] * eye[:, None, :]).reshape(HD, H)


def _perm96(w):
    return w.reshape(-1, 3, 2, 16).transpose(0, 1, 3, 2).reshape(-1, FBW)


def _wbig(fc, albd):
    wn = jnp.concatenate([fc, fc @ albd, jnp.zeros((F, 24), jnp.float32)], axis=1)
    return _perm96(wn)


def _cbias():
    nat = jnp.concatenate([jnp.zeros((1, 72), jnp.float32),
                           jnp.full((1, 24), -1e30, jnp.float32)], axis=1)
    return _perm96(nat)


def _pad_idx(v):
    p = jnp.concatenate([v, jnp.full((EP - E,), NP - 1, jnp.int32)])
    return p.reshape(EP // CH, CH)


def kernel(h, edge_index0, edge_index1, fc0, attn_l0, attn_r0, bias0,
           fc1, attn_l1, attn_r1, bias1, Ws1, bs1, Ws2, Wp, bp):
    h_pad = jnp.pad(h, ((0, NP - N), (0, 0)))
    cbias = _cbias()
    fb0, fb1, er0, er1, elm0, elm1 = _pre_call(
        h_pad,
        _wbig(fc0, _blockdiag(attn_l0)), fc0 @ _blockdiag(attn_l0),
        fc0 @ _blockdiag(attn_r0),
        _wbig(fc1, _blockdiag(attn_l1)), fc1 @ _blockdiag(attn_l1),
        fc1 @ _blockdiag(attn_r1),
        cbias)

    def _pack(fb):
        return jax.lax.bitcast_convert_type(
            fb.reshape(NP, FBW // 2, 2), jnp.int32)

    o0, o1 = _sc_edge(
        _pack(fb0), er0, jnp.pad(elm0.reshape(8), (0, 8)),
        _pad_idx(edge_index0[0]), _pad_idx(edge_index0[1]),
        _pack(fb1), er1, jnp.pad(elm1.reshape(8), (0, 8)),
        _pad_idx(edge_index1[0]), _pad_idx(edge_index1[1]))

    z0, z1, s0, s1 = _mid_call(
        o0, o0, o1, o1,
        bias0.reshape(1, HD), bias1.reshape(1, HD),
        Ws1, bs1.reshape(1, SEM), Ws2.reshape(1, SEM))
    ws = jnp.concatenate([s0, s1], axis=0)
    out_full = _fin_call(ws, z0, z1, Wp, bp.reshape(1, OUT))
    return out_full[:N]


# speedup vs baseline: 110.3302x; 1.0566x over previous
import functools

import jax
import jax.numpy as jnp
from jax import lax
from jax.experimental import pallas as pl
from jax.experimental.pallas import tpu as pltpu
from jax.experimental.pallas import tpu_sc as plsc

N = 10000
F = 128
H = 8
D = 8
HD = 64
SEM = 128
OUT = 16
NP = 10240
E = 320000
EP = 327680
NTILES = 32
EPT = EP // NTILES
CH = 128
NCHUNK = EPT // CH
FBW = 96
RW = 80
BN = 640
GRID = NP // BN
NB = 4


def _tc_pre(h_ref, wb0_ref, fcl0_ref, fcr0_ref, wb1_ref, fcl1_ref, fcr1_ref,
            cb_ref, fb0_ref, fb1_ref, er0_ref, er1_ref, elm0_ref, elm1_ref):
    blk = h_ref[...]
    first = pl.program_id(0) == 0
    for (wb, fcl, fcr, fb_out, er_out, elm) in (
            (wb0_ref, fcl0_ref, fcr0_ref, fb0_ref, er0_ref, elm0_ref),
            (wb1_ref, fcl1_ref, fcr1_ref, fb1_ref, er1_ref, elm1_ref)):
        fb = jnp.dot(blk, wb[...], preferred_element_type=jnp.float32) + cb_ref[...]
        fb_out[...] = fb.astype(jnp.bfloat16)
        el = jnp.dot(blk, fcl[...], preferred_element_type=jnp.float32)
        er = jnp.dot(blk, fcr[...], preferred_element_type=jnp.float32)
        er_out[...] = jnp.concatenate([er, jnp.zeros((BN, 8), jnp.float32)], axis=1)
        m = jnp.max(el, axis=0, keepdims=True)

        @pl.when(first)
        def _():
            elm[...] = m

        @pl.when(jnp.logical_not(first))
        def _():
            elm[...] = jnp.maximum(elm[...], m)


_pre_call = pl.pallas_call(
    _tc_pre,
    grid=(GRID,),
    in_specs=[
        pl.BlockSpec((BN, F), lambda i: (i, 0)),
        pl.BlockSpec((F, FBW), lambda i: (0, 0)),
        pl.BlockSpec((F, H), lambda i: (0, 0)),
        pl.BlockSpec((F, H), lambda i: (0, 0)),
        pl.BlockSpec((F, FBW), lambda i: (0, 0)),
        pl.BlockSpec((F, H), lambda i: (0, 0)),
        pl.BlockSpec((F, H), lambda i: (0, 0)),
        pl.BlockSpec((1, FBW), lambda i: (0, 0)),
    ],
    out_specs=[
        pl.BlockSpec((BN, FBW), lambda i: (i, 0)),
        pl.BlockSpec((BN, FBW), lambda i: (i, 0)),
        pl.BlockSpec((BN, 16), lambda i: (i, 0)),
        pl.BlockSpec((BN, 16), lambda i: (i, 0)),
        pl.BlockSpec((1, 8), lambda i: (0, 0)),
        pl.BlockSpec((1, 8), lambda i: (0, 0)),
    ],
    out_shape=[
        jax.ShapeDtypeStruct((NP, FBW), jnp.bfloat16),
        jax.ShapeDtypeStruct((NP, FBW), jnp.bfloat16),
        jax.ShapeDtypeStruct((NP, 16), jnp.float32),
        jax.ShapeDtypeStruct((NP, 16), jnp.float32),
        jax.ShapeDtypeStruct((1, 8), jnp.float32),
        jax.ShapeDtypeStruct((1, 8), jnp.float32),
    ],
)


def _dyn_gather(x, idx):
    return lax.gather(
        x, idx[:, None],
        lax.GatherDimensionNumbers(offset_dims=(), collapsed_slice_dims=(0,),
                                   start_index_map=(0,)),
        (1,), mode=lax.GatherScatterMode.PROMISE_IN_BOUNDS)


_sc_mesh = plsc.VectorSubcoreMesh(core_axis_name="c", subcore_axis_name="s")


@functools.partial(
    pl.kernel,
    mesh=_sc_mesh,
    compiler_params=pltpu.CompilerParams(use_tc_tiling_on_sc=False),
    out_type=[
        jax.ShapeDtypeStruct((2 * NP, RW), jnp.float32),
        jax.ShapeDtypeStruct((2 * NP, RW), jnp.float32),
    ],
    scratch_types=(
        [pltpu.VMEM((NCHUNK, CH), jnp.int32)] * 2
        + [pltpu.VMEM((CH, FBW // 2), jnp.int32)] * NB
        + [pltpu.VMEM((CH, 16), jnp.float32)] * NB
        + [pltpu.VMEM((CH, RW), jnp.float32)] * 2
        + [pltpu.VMEM((16,), jnp.float32)]
        + [pltpu.VMEM_SHARED((NP, RW), jnp.float32)]
        + [pltpu.SemaphoreType.DMA] * (2 * NB + 2)
    ),
)
def _sc_edge(fb0, er0, elm0, src0, dst0, fb1, er1, elm1, src1, dst1,
             out0, out1, *sc):
    sidx, didx = sc[0], sc[1]
    gb = sc[2:2 + NB]
    eb = sc[2 + NB:2 + 2 * NB]
    mb = sc[2 + 2 * NB:4 + 2 * NB]
    elmb = sc[4 + 2 * NB]
    accum = sc[5 + 2 * NB]
    semf = sc[6 + 2 * NB:6 + 3 * NB]
    seme = sc[6 + 3 * NB:6 + 4 * NB]
    sems = sc[6 + 4 * NB:8 + 4 * NB]

    cid = lax.axis_index("c")
    sid = lax.axis_index("s")
    iota = lax.iota(jnp.int32, 16)
    lo8 = iota & 7
    hi = jnp.right_shift(iota, 3)
    zero16 = jnp.zeros((16,), jnp.float32)

    def zrow(i, carry):
        for k in range(RW // 16):
            mb[0][i, pl.ds(16 * k, 16)] = zero16
        return carry

    tile_rows = NP // 16
    tchunk0 = (cid * 16 + sid) * NCHUNK

    for (ftab, ertab, elmv, srcv, dstv, outv) in (
            (fb0, er0, elm0, src0, dst0, out0),
            (fb1, er1, elm1, src1, dst1, out1)):
        pltpu.sync_copy(srcv.at[pl.ds(tchunk0, NCHUNK)], sidx)
        pltpu.sync_copy(dstv.at[pl.ds(tchunk0, NCHUNK)], didx)
        pltpu.sync_copy(elmv, elmb)
        emv = _dyn_gather(elmb[...], lo8)
        lax.fori_loop(0, CH, zrow, 0)
        for r in range(tile_rows // CH):
            pltpu.sync_copy(mb[0], accum.at[pl.ds(sid * tile_rows + r * CH, CH)])
        plsc.subcore_barrier()

        def fire(slot, j):
            pltpu.async_copy(ftab.at[sidx.at[j]], gb[slot], semf[slot])
            pltpu.async_copy(ertab.at[didx.at[j]], eb[slot], seme[slot])

        for b in range(NB):
            fire(b, b)

        def edge_block(g, e, mbuf):
            def edge(i, icarry):
                w0 = g[i, pl.ds(0, 16)]
                w1 = g[i, pl.ds(16, 16)]
                w2 = g[i, pl.ds(32, 16)]
                msk = jnp.int32(-65536)
                f0l = lax.bitcast_convert_type(lax.shift_left(w0, 16), jnp.float32)
                f0h = lax.bitcast_convert_type(w0 & msk, jnp.float32)
                f1l = lax.bitcast_convert_type(lax.shift_left(w1, 16), jnp.float32)
                f1h = lax.bitcast_convert_type(w1 & msk, jnp.float32)
                a = lax.bitcast_convert_type(lax.shift_left(w2, 16), jnp.float32)
                bv = e[i, :]
                t = a + bv
                lr = jnp.maximum(t, 0.2 * t)
                u = bv + emv
                cv = jnp.maximum(u, 0.2 * u)
                ex = jnp.exp(lr - cv)
                mbuf[i, pl.ds(HD, 16)] = ex
                mbuf[i, pl.ds(0, 16)] = f0l * _dyn_gather(ex, hi)
                mbuf[i, pl.ds(16, 16)] = f0h * _dyn_gather(ex, 2 + hi)
                mbuf[i, pl.ds(32, 16)] = f1l * _dyn_gather(ex, 4 + hi)
                mbuf[i, pl.ds(48, 16)] = f1h * _dyn_gather(ex, 6 + hi)
                return icarry

            lax.fori_loop(0, CH, edge, 0, unroll=4)

        def macro(jj, carry):
            for b in range(NB):
                j = jj * NB + b
                par = b % 2
                pltpu.make_async_copy(ftab.at[sidx.at[j]], gb[b], semf[b]).wait()
                pltpu.make_async_copy(ertab.at[didx.at[j]], eb[b], seme[b]).wait()

                @pl.when(j >= 2)
                def _():
                    pltpu.make_async_copy(
                        mb[par], accum.at[didx.at[j - 2]], sems[par]).wait()

                edge_block(gb[b], eb[b], mb[par])
                pltpu.async_copy(mb[par], accum.at[didx.at[j]], sems[par],
                                 add=True)

                @pl.when(j + NB < NCHUNK)
                def _():
                    fire(b, j + NB)
            return carry

        lax.fori_loop(0, NCHUNK // NB, macro, 0)
        for par in range(2):
            pltpu.make_async_copy(
                mb[par], accum.at[didx.at[NCHUNK - 2 + par]], sems[par]).wait()
        plsc.subcore_barrier()
        for r in range(tile_rows // CH):
            row0 = sid * tile_rows + r * CH
            pltpu.sync_copy(accum.at[pl.ds(row0, CH)],
                            outv.at[pl.ds(cid * NP + row0, CH)])
        plsc.subcore_barrier()


def _tc_mid(o0a_ref, o0b_ref, o1a_ref, o1b_ref, b0_ref, b1_ref,
            ws1_ref, bs1_ref, ws2_ref, z0_ref, z1_ref, s0_ref, s1_ref):
    i = pl.program_id(0)
    first = i == 0
    rows = i * BN + lax.broadcasted_iota(jnp.int32, (BN, 1), 0)
    mask = (rows < N).astype(jnp.float32)

    for (oa, ob, b_ref, z_ref, s_ref) in ((o0a_ref, o0b_ref, b0_ref, z0_ref, s0_ref),
                                          (o1a_ref, o1b_ref, b1_ref, z1_ref, s1_ref)):
        num = oa[:, 0:HD] + ob[:, 0:HD]
        den = oa[:, HD:HD + 8] + ob[:, HD:HD + 8] + 1e-9
        r = num.reshape(BN, H, D) / den[:, :, None] + b_ref[...].reshape(1, H, D)
        z = jnp.where(r > 0, r, jnp.exp(jnp.minimum(r, 0.0)) - 1.0)
        z = z.reshape(BN, HD)
        z_ref[...] = z
        t = jnp.tanh(jnp.dot(z, ws1_ref[...], preferred_element_type=jnp.float32)
                     + bs1_ref[...])
        w = jnp.sum(t * ws2_ref[...], axis=1, keepdims=True)
        s = jnp.sum(w * mask)
        sv = jnp.full((1, 8), s, jnp.float32)

        @pl.when(first)
        def _():
            s_ref[...] = sv

        @pl.when(jnp.logical_not(first))
        def _():
            s_ref[...] = s_ref[...] + sv


_mid_call = pl.pallas_call(
    _tc_mid,
    grid=(GRID,),
    in_specs=[
        pl.BlockSpec((BN, RW), lambda i: (i, 0)),
        pl.BlockSpec((BN, RW), lambda i: (GRID + i, 0)),
        pl.BlockSpec((BN, RW), lambda i: (i, 0)),
        pl.BlockSpec((BN, RW), lambda i: (GRID + i, 0)),
        pl.BlockSpec((1, HD), lambda i: (0, 0)),
        pl.BlockSpec((1, HD), lambda i: (0, 0)),
        pl.BlockSpec((HD, SEM), lambda i: (0, 0)),
        pl.BlockSpec((1, SEM), lambda i: (0, 0)),
        pl.BlockSpec((1, SEM), lambda i: (0, 0)),
    ],
    out_specs=[
        pl.BlockSpec((BN, HD), lambda i: (i, 0)),
        pl.BlockSpec((BN, HD), lambda i: (i, 0)),
        pl.BlockSpec((1, 8), lambda i: (0, 0)),
        pl.BlockSpec((1, 8), lambda i: (0, 0)),
    ],
    out_shape=[
        jax.ShapeDtypeStruct((NP, HD), jnp.float32),
        jax.ShapeDtypeStruct((NP, HD), jnp.float32),
        jax.ShapeDtypeStruct((1, 8), jnp.float32),
        jax.ShapeDtypeStruct((1, 8), jnp.float32),
    ],
)


def _tc_fin(ws_ref, z0_ref, z1_ref, wp_ref, bp_ref, out_ref):
    wn = ws_ref[...] * (1.0 / N)
    m = jnp.max(wn)
    e = jnp.exp(wn - m)
    r0 = e[0:1, 0:1]
    r1 = e[1:2, 0:1]
    tot = r0 + r1
    b0 = r0 / tot
    b1 = r1 / tot
    hout = b0 * z0_ref[...] + b1 * z1_ref[...]
    out_ref[...] = (jnp.dot(hout, wp_ref[...], preferred_element_type=jnp.float32)
                    + bp_ref[...])


_fin_call = pl.pallas_call(
    _tc_fin,
    grid=(GRID,),
    in_specs=[
        pl.BlockSpec((2, 8), lambda i: (0, 0)),
        pl.BlockSpec((BN, HD), lambda i: (i, 0)),
        pl.BlockSpec((BN, HD), lambda i: (i, 0)),
        pl.BlockSpec((HD, OUT), lambda i: (0, 0)),
        pl.BlockSpec((1, OUT), lambda i: (0, 0)),
    ],
    out_specs=pl.BlockSpec((BN, OUT), lambda i: (i, 0)),
    out_shape=jax.ShapeDtypeStruct((NP, OUT), jnp.float32),
)


def _blockdiag(a):
    eye = jnp.eye(H, dtype=a.dtype)
    return (a[:, :, None] * eye[:, None, :]).reshape(HD, H)


def _perm96(w):
    return w.reshape(-1, 3, 2, 16).transpose(0, 1, 3, 2).reshape(-1, FBW)


def _wbig(fc, albd):
    wn = jnp.concatenate([fc, fc @ albd, jnp.zeros((F, 24), jnp.float32)], axis=1)
    return _perm96(wn)


def _cbias():
    nat = jnp.concatenate([jnp.zeros((1, 72), jnp.float32),
                           jnp.full((1, 24), -1e30, jnp.float32)], axis=1)
    return _perm96(nat)


def _pad_idx(v):
    p = jnp.concatenate([v, jnp.full((EP - E,), NP - 1, jnp.int32)])
    return p.reshape(EP // CH, CH)


def kernel(h, edge_index0, edge_index1, fc0, attn_l0, attn_r0, bias0,
           fc1, attn_l1, attn_r1, bias1, Ws1, bs1, Ws2, Wp, bp):
    h_pad = jnp.pad(h, ((0, NP - N), (0, 0)))
    cbias = _cbias()
    fb0, fb1, er0, er1, elm0, elm1 = _pre_call(
        h_pad,
        _wbig(fc0, _blockdiag(attn_l0)), fc0 @ _blockdiag(attn_l0),
        fc0 @ _blockdiag(attn_r0),
        _wbig(fc1, _blockdiag(attn_l1)), fc1 @ _blockdiag(attn_l1),
        fc1 @ _blockdiag(attn_r1),
        cbias)

    def _pack(fb):
        return jax.lax.bitcast_convert_type(
            fb.reshape(NP, FBW // 2, 2), jnp.int32)

    o0, o1 = _sc_edge(
        _pack(fb0), er0, jnp.pad(elm0.reshape(8), (0, 8)),
        _pad_idx(edge_index0[0]), _pad_idx(edge_index0[1]),
        _pack(fb1), er1, jnp.pad(elm1.reshape(8), (0, 8)),
        _pad_idx(edge_index1[0]), _pad_idx(edge_index1[1]))

    z0, z1, s0, s1 = _mid_call(
        o0, o0, o1, o1,
        bias0.reshape(1, HD), bias1.reshape(1, HD),
        Ws1, bs1.reshape(1, SEM), Ws2.reshape(1, SEM))
    ws = jnp.concatenate([s0, s1], axis=0)
    out_full = _fin_call(ws, z0, z1, Wp, bp.reshape(1, OUT))
    return out_full[:N]
